# Initial kernel scaffold; baseline (speedup 1.0000x reference)
import jax
import jax.numpy as jnp
from jax.experimental import pallas as pl


def kernel(feat, segment_ids):
    raise NotImplementedError("write your pallas kernel here")


# speedup vs baseline: 6.0437x; 6.0437x over previous
import functools

import jax
import jax.numpy as jnp
from jax import lax
from jax.experimental import pallas as pl
from jax.experimental.pallas import tpu as pltpu
from jax.experimental.pallas import tpu_sc as plsc

NUM_SEGMENTS = 256
E = 320000
D = 128

NC = 2
NS = 16
NW = NC * NS
ROWS_PER_TILE = E // NW
BLK = 128
NFULL = ROWS_PER_TILE // BLK
TAIL = ROWS_PER_TILE - NFULL * BLK
IDROWS = NFULL + 2
SEGS_PER_TILE = NUM_SEGMENTS // NS

_mesh = plsc.VectorSubcoreMesh(core_axis_name="c", subcore_axis_name="s")


def _seg_sum_body(feat, ids2, out, fbuf, tbuf, ibuf, zbuf, acc,
                  sem0, sem1, semi):
    c = lax.axis_index("c")
    s = lax.axis_index("s")
    sems = (sem0, sem1)
    w = s * NC + c
    base = w * ROWS_PER_TILE

    pltpu.async_copy(ids2.at[pl.ds(w * IDROWS, IDROWS)], ibuf, semi)
    pltpu.async_copy(
        feat.at[pl.ds(base + NFULL * BLK, TAIL), :],
        tbuf.at[pl.ds(0, TAIL)], semi)

    zero16 = jnp.zeros((16,), jnp.float32)
    for r in range(SEGS_PER_TILE):
        for j in range(D // 16):
            zbuf[r, pl.ds(j * 16, 16)] = zero16
    for r in range(TAIL, BLK):
        for j in range(D // 16):
            tbuf[r, pl.ds(j * 16, 16)] = zero16

    pltpu.sync_copy(zbuf, acc.at[pl.ds(s * SEGS_PER_TILE, SEGS_PER_TILE)])
    plsc.subcore_barrier()

    def start_block(i, b):
        pltpu.async_copy(
            feat.at[pl.ds(base + i * BLK, BLK), :], fbuf.at[b], sems[b])

    def wait_block(b):
        pltpu.make_async_copy(
            feat.at[pl.ds(0, BLK), :], fbuf.at[b], sems[b]).wait()

    start_block(0, 0)
    start_block(1, 1)

    pltpu.make_async_copy(
        ids2.at[pl.ds(0, IDROWS)], ibuf, semi).wait()
    pltpu.make_async_copy(
        feat.at[pl.ds(0, TAIL), :], tbuf.at[pl.ds(0, TAIL)], semi).wait()

    def loop_body(iv, carry):
        for b in range(2):
            i = 2 * iv + b
            wait_block(b)
            pltpu.sync_copy(fbuf.at[b], acc.at[ibuf.at[i]], add=True)

            @pl.when(i + 2 < NFULL)
            def _prefetch():
                start_block(i + 2, b)
        return carry

    lax.fori_loop(0, NFULL // 2, loop_body, None)

    pltpu.sync_copy(tbuf, acc.at[ibuf.at[NFULL]], add=True)

    plsc.subcore_barrier()
    seg0 = s * SEGS_PER_TILE
    pltpu.sync_copy(
        acc.at[pl.ds(seg0, SEGS_PER_TILE)],
        out.at[c, pl.ds(seg0, SEGS_PER_TILE), :])


_seg_sum = pl.kernel(
    _seg_sum_body,
    out_type=jax.ShapeDtypeStruct((NC, NUM_SEGMENTS, D), jnp.float32),
    mesh=_mesh,
    scratch_types=[
        pltpu.VMEM((2, BLK, D), jnp.float32),
        pltpu.VMEM((BLK, D), jnp.float32),
        pltpu.VMEM((IDROWS, BLK), jnp.int32),
        pltpu.VMEM((SEGS_PER_TILE, D), jnp.float32),
        pltpu.VMEM_SHARED((NUM_SEGMENTS, D), jnp.float32),
        pltpu.SemaphoreType.DMA,
        pltpu.SemaphoreType.DMA,
        pltpu.SemaphoreType.DMA,
    ],
)


def _combine_body(p_ref, o_ref):
    o_ref[...] = p_ref[0] + p_ref[1]


_combine = pl.pallas_call(
    _combine_body,
    out_shape=jax.ShapeDtypeStruct((NUM_SEGMENTS, D), jnp.float32),
)


def kernel(feat, segment_ids):
    ids2 = jnp.pad(
        segment_ids.reshape(NW, ROWS_PER_TILE),
        ((0, 0), (0, IDROWS * BLK - ROWS_PER_TILE)),
    ).reshape(NW * IDROWS, BLK)
    partials = _seg_sum(feat, ids2)
    return _combine(partials)
